# baseline (speedup 1.0000x reference)
import jax
import jax.numpy as jnp
from jax import lax
from jax.experimental import pallas as pl
from jax.experimental.pallas import tpu as pltpu
from jax.experimental.pallas import tpu_sc as plsc

N = 50000
E = 800000
R = 64
ORDER = 3
FLOW_LEN = 100
BS = N // FLOW_LEN
C = 12

HALF = N // 2
NTILE = 16
CHUNK = 128
EROWS = E // CHUNK
ROWS_PT = EROWS // NTILE
EXTRA = EROWS - ROWS_PT * NTILE
DUMMY_SPAN = 1024
ZROWS = 1627
ACC_ROWS = NTILE * ZROWS
WB = 1560

BLK = 2000
GRID = N // BLK
GROUPS = BLK // FLOW_LEN


def _segsum_body(h_hbm, src_hbm, dst_hbm, zeros_hbm, agg_hbm,
                 sb0, sb1, db0, db1, lb0, lb1, rb0, rb1, acc,
                 sg0, sg1, si0, si1, ss0, ss1):
    cid = lax.axis_index("c")
    sid = lax.axis_index("s")
    core_base = cid * HALF

    pltpu.sync_copy(zeros_hbm, acc.at[pl.ds(sid * ZROWS, ZROWS)])
    plsc.subcore_barrier()

    srcb = [sb0, sb1]
    dstb = [db0, db1]
    dstl = [lb0, lb1]
    rows = [rb0, rb1]
    sg = [sg0, sg1]
    si = [si0, si1]
    ss = [ss0, ss1]
    base_row = sid * ROWS_PT
    nloc = jnp.where(sid < EXTRA, ROWS_PT + 1, ROWS_PT)

    def row_of(j):
        return jnp.where(j < ROWS_PT, base_row + j, NTILE * ROWS_PT + sid)

    def sdesc(b, row):
        return pltpu.make_async_copy(src_hbm.at[row], srcb[b], si[b])

    def ddesc(b, row):
        return pltpu.make_async_copy(dst_hbm.at[row], dstb[b], si[b])

    def gdesc(b):
        return pltpu.make_async_copy(h_hbm.at[srcb[b]], rows[b], sg[b])

    def scdesc(b):
        return pltpu.make_async_copy(rows[b], acc.at[dstl[b]], ss[b])

    def compute_dstl(b):
        for k in range(CHUNK // 16):
            dv = dstb[b][pl.ds(16 * k, 16)]
            loc = dv - core_base
            ok = (loc >= 0) & (loc < HALF)
            alt = HALF + (dv & (DUMMY_SPAN - 1))
            dstl[b][pl.ds(16 * k, 16)] = jnp.where(ok, loc, alt)

    pltpu.sync_copy(src_hbm.at[base_row], srcb[0])
    pltpu.sync_copy(dst_hbm.at[base_row], dstb[0])
    compute_dstl(0)
    gdesc(0).start()
    sdesc(1, row_of(1)).start()
    ddesc(1, row_of(1)).start()

    def step(j, b):
        bn = 1 - b
        nr = row_of(j + 1)

        @pl.when(j + 1 < nloc)
        def _():
            sdesc(bn, nr).wait()
            ddesc(bn, nr).wait()

        gdesc(b).wait()

        @pl.when(j >= 1)
        def _():
            scdesc(bn).wait()

        @pl.when(j + 1 < nloc)
        def _():
            compute_dstl(bn)
            gdesc(bn).start()

        @pl.when(j + 2 < nloc)
        def _():
            r2 = row_of(j + 2)
            sdesc(b, r2).start()
            ddesc(b, r2).start()

        scdesc(b).start(add=True)

    def pair(jj, carry):
        j = jj * 2
        step(j, 0)
        step(j + 1, 1)
        return carry

    lax.fori_loop(0, ROWS_PT // 2, pair, 0)

    scdesc(1).wait()

    @pl.when(nloc > ROWS_PT)
    def _():
        gdesc(0).wait()
        pltpu.sync_copy(rows[0], acc.at[dstl[0]], add=True)

    plsc.subcore_barrier()

    @pl.when(sid < NTILE - 1)
    def _():
        pltpu.sync_copy(acc.at[pl.ds(sid * WB, WB)],
                        agg_hbm.at[pl.ds(core_base + sid * WB, WB)])

    @pl.when(sid == NTILE - 1)
    def _():
        rest = HALF - (NTILE - 1) * WB
        pltpu.sync_copy(acc.at[pl.ds((NTILE - 1) * WB, rest)],
                        agg_hbm.at[pl.ds(core_base + (NTILE - 1) * WB, rest)])


def _segsum(table, src2, dst2, zeros, width=R):
    return pl.kernel(
        _segsum_body,
        mesh=plsc.VectorSubcoreMesh(core_axis_name="c", subcore_axis_name="s"),
        compiler_params=pltpu.CompilerParams(use_tc_tiling_on_sc=False),
        out_type=jax.ShapeDtypeStruct((N, width), jnp.float32),
        scratch_types=[
            pltpu.VMEM((CHUNK,), jnp.int32),
            pltpu.VMEM((CHUNK,), jnp.int32),
            pltpu.VMEM((CHUNK,), jnp.int32),
            pltpu.VMEM((CHUNK,), jnp.int32),
            pltpu.VMEM((CHUNK,), jnp.int32),
            pltpu.VMEM((CHUNK,), jnp.int32),
            pltpu.VMEM((CHUNK, width), jnp.float32),
            pltpu.VMEM((CHUNK, width), jnp.float32),
            pltpu.VMEM_SHARED((ACC_ROWS, width), jnp.float32),
            pltpu.SemaphoreType.DMA,
            pltpu.SemaphoreType.DMA,
            pltpu.SemaphoreType.DMA,
            pltpu.SemaphoreType.DMA,
            pltpu.SemaphoreType.DMA,
            pltpu.SemaphoreType.DMA,
        ],
    )(table, src2, dst2, zeros)


def _seed_body(f_ref, w_ref, b_ref, h_ref):
    h_ref[...] = f_ref[...] * w_ref[...] + b_ref[...]


def _pass1a_body(f_ref, s_ref, a_ref, sa_ref, qa_ref):
    a = f_ref[...] + s_ref[...]
    a_ref[...] = a
    sa_ref[...] = jnp.sum(a, axis=(0, 1), keepdims=True)[None]
    qa_ref[...] = jnp.sum(a * a, axis=(0, 1), keepdims=True)[None]


def _pass2a_body(a_ref, w_ref, g0_ref, l0_ref, gb0_ref, lb0_ref,
                 sa_ref, qa_ref, bng_ref, bnb_ref, h_ref, ro_ref):
    wv = jnp.dot(jnp.dot(w_ref[...], g0_ref[...], preferred_element_type=jnp.float32),
                 l0_ref[...], preferred_element_type=jnp.float32)
    kv = jnp.dot(gb0_ref[...], l0_ref[...], preferred_element_type=jnp.float32) + lb0_ref[...]
    am = jnp.sum(sa_ref[...], axis=(0, 1), keepdims=False)[None] * (1.0 / N)
    av = jnp.sum(qa_ref[...], axis=(0, 1), keepdims=False)[None] * (1.0 / N) - am * am
    meanz = am * wv + kv
    varz = av * (wv * wv)
    inv = lax.rsqrt(varz + 1e-5)
    scale = bng_ref[...] * inv
    shift = bnb_ref[...] - meanz * scale
    z = jnp.dot(a_ref[...], wv, preferred_element_type=jnp.float32) + kv
    hn = jnp.maximum(z * scale + shift, 0.0)
    h_ref[...] = hn
    ro_ref[...] = hn.reshape(GROUPS, FLOW_LEN, R).sum(axis=1)[None]


def _pass1_body(h_ref, agg_ref, g_ref, gb_ref, l_ref, lb_ref, z_ref, s_ref, q_ref):
    x = h_ref[...] + agg_ref[...]
    z = jnp.dot(x, g_ref[...], preferred_element_type=jnp.float32) + gb_ref[...]
    z = jnp.dot(z, l_ref[...], preferred_element_type=jnp.float32) + lb_ref[...]
    z_ref[...] = z

    @pl.when(pl.program_id(0) == 0)
    def _():
        s_ref[...] = jnp.zeros_like(s_ref)
        q_ref[...] = jnp.zeros_like(q_ref)

    s_ref[...] += jnp.sum(z, axis=0, keepdims=True)
    q_ref[...] += jnp.sum(z * z, axis=0, keepdims=True)


def _pass2_body(z_ref, s_ref, q_ref, g_ref, b_ref, h_ref, ro_ref):
    mean = s_ref[...] * (1.0 / N)
    var = q_ref[...] * (1.0 / N) - mean * mean
    inv = lax.rsqrt(var + 1e-5)
    scale = g_ref[...] * inv
    shift = b_ref[...] - mean * scale
    hn = jnp.maximum(z_ref[...] * scale + shift, 0.0)
    h_ref[...] = hn
    ro_ref[...] = hn.reshape(GROUPS, FLOW_LEN, R).sum(axis=1)[None]


def _cls_body(r0_ref, r1_ref, r2_ref, w0_ref, w1_ref, w2_ref, b_ref, y_ref):
    y = jnp.dot(r0_ref[...], w0_ref[...], preferred_element_type=jnp.float32)
    y += jnp.dot(r1_ref[...], w1_ref[...], preferred_element_type=jnp.float32)
    y += jnp.dot(r2_ref[...], w2_ref[...], preferred_element_type=jnp.float32)
    y_ref[...] = y + b_ref[...]


def kernel(feats, edge_index, W_seq, b_seq, gin_W, gin_b, lin_W, lin_b, bn_g, bn_b, cls_W, cls_b):
    f32 = jnp.float32
    src = edge_index[0].astype(jnp.int32)
    dst = edge_index[1].astype(jnp.int32)
    src2 = src.reshape(EROWS, CHUNK)
    dst2 = dst.reshape(EROWS, CHUNK)
    zeros = jnp.zeros((ZROWS, R), f32)
    zeros16 = jnp.zeros((ZROWS, 16), f32)

    h = pl.pallas_call(
        _seed_body,
        grid=(GRID,),
        in_specs=[
            pl.BlockSpec((BLK, 1), lambda i: (i, 0)),
            pl.BlockSpec((1, R), lambda i: (0, 0)),
            pl.BlockSpec((1, R), lambda i: (0, 0)),
        ],
        out_specs=pl.BlockSpec((BLK, R), lambda i: (i, 0)),
        out_shape=jax.ShapeDtypeStruct((N, R), f32),
    )(feats.reshape(N, 1), W_seq, b_seq.reshape(1, R))

    ros = []
    for i in range(0, ORDER):
        agg = _segsum(h, src2, dst2, zeros)

        z, s, q = pl.pallas_call(
            _pass1_body,
            grid=(GRID,),
            in_specs=[
                pl.BlockSpec((BLK, R), lambda i: (i, 0)),
                pl.BlockSpec((BLK, R), lambda i: (i, 0)),
                pl.BlockSpec((R, R), lambda i: (0, 0)),
                pl.BlockSpec((1, R), lambda i: (0, 0)),
                pl.BlockSpec((R, R), lambda i: (0, 0)),
                pl.BlockSpec((1, R), lambda i: (0, 0)),
            ],
            out_specs=[
                pl.BlockSpec((BLK, R), lambda i: (i, 0)),
                pl.BlockSpec((1, R), lambda i: (0, 0)),
                pl.BlockSpec((1, R), lambda i: (0, 0)),
            ],
            out_shape=[
                jax.ShapeDtypeStruct((N, R), f32),
                jax.ShapeDtypeStruct((1, R), f32),
                jax.ShapeDtypeStruct((1, R), f32),
            ],
        )(h, agg, gin_W[i], gin_b[i].reshape(1, R), lin_W[i], lin_b[i].reshape(1, R))

        h, ro = pl.pallas_call(
            _pass2_body,
            grid=(GRID,),
            in_specs=[
                pl.BlockSpec((BLK, R), lambda i: (i, 0)),
                pl.BlockSpec((1, R), lambda i: (0, 0)),
                pl.BlockSpec((1, R), lambda i: (0, 0)),
                pl.BlockSpec((1, R), lambda i: (0, 0)),
                pl.BlockSpec((1, R), lambda i: (0, 0)),
            ],
            out_specs=[
                pl.BlockSpec((BLK, R), lambda i: (i, 0)),
                pl.BlockSpec((1, GROUPS, R), lambda i: (i, 0, 0)),
            ],
            out_shape=[
                jax.ShapeDtypeStruct((N, R), f32),
                jax.ShapeDtypeStruct((GRID, GROUPS, R), f32),
            ],
        )(z, s, q, bn_g[i].reshape(1, R), bn_b[i].reshape(1, R))
        ros.append(ro.reshape(BS, R))

    wp = jnp.pad(cls_W, ((0, 0), (0, 128 - C)))
    bp = jnp.pad(cls_b, (0, 128 - C)).reshape(1, 128)
    y = pl.pallas_call(
        _cls_body,
        out_shape=jax.ShapeDtypeStruct((BS, 128), f32),
    )(ros[0], ros[1], ros[2], wp[0:R], wp[R:2 * R], wp[2 * R:3 * R], bp)
    return y[:, :C]


# speedup vs baseline: 1.1278x; 1.1278x over previous
import jax
import jax.numpy as jnp
from jax import lax
from jax.experimental import pallas as pl
from jax.experimental.pallas import tpu as pltpu
from jax.experimental.pallas import tpu_sc as plsc

N = 50000
E = 800000
R = 64
ORDER = 3
FLOW_LEN = 100
BS = N // FLOW_LEN
C = 12

HW = R // 2
NTILE = 16
CHUNK = 128
EROWS = E // CHUNK
ROWS_PT = EROWS // NTILE
EXTRA = EROWS - ROWS_PT * NTILE
Z32 = 3128
ACC_ROWS = NTILE * Z32
WB = 3120

BLK = 2000
GRID = N // BLK
GROUPS = BLK // FLOW_LEN


def _segsum_body(h_hbm, src_hbm, dst_hbm, zeros_hbm, agg_hbm,
                 sb0, sb1, db0, db1, lb0, lb1, rb0, rb1, acc,
                 sg0, sg1, si0, si1):
    cid = lax.axis_index("c")
    sid = lax.axis_index("s")

    pltpu.sync_copy(zeros_hbm, acc.at[pl.ds(sid * Z32, Z32)])
    plsc.subcore_barrier()

    srcb = [sb0, sb1]
    dstb = [db0, db1]
    gidx = [lb0, lb1]
    rows = [rb0, rb1]
    sg = [sg0, sg1]
    si = [si0, si1]
    base_row = sid * ROWS_PT
    nloc = jnp.where(sid < EXTRA, ROWS_PT + 1, ROWS_PT)

    def row_of(j):
        return jnp.where(j < ROWS_PT, base_row + j, NTILE * ROWS_PT + sid)

    def sdesc(b, row):
        return pltpu.make_async_copy(src_hbm.at[row], srcb[b], si[b])

    def ddesc(b, row):
        return pltpu.make_async_copy(dst_hbm.at[row], dstb[b], si[b])

    def gdesc(b):
        return pltpu.make_async_copy(h_hbm.at[gidx[b]], rows[b], sg[b])

    def compute_gidx(b):
        for k in range(CHUNK // 16):
            sv = srcb[b][pl.ds(16 * k, 16)]
            gidx[b][pl.ds(16 * k, 16)] = sv + sv + cid

    pltpu.sync_copy(src_hbm.at[base_row], srcb[0])
    pltpu.sync_copy(dst_hbm.at[base_row], dstb[0])
    compute_gidx(0)
    gdesc(0).start()
    sdesc(1, row_of(1)).start()
    ddesc(1, row_of(1)).start()

    def step(j, b):
        bn = 1 - b
        nr = row_of(j + 1)

        @pl.when(j + 1 < nloc)
        def _():
            sdesc(bn, nr).wait()
            ddesc(bn, nr).wait()
            compute_gidx(bn)

        gdesc(b).wait()

        @pl.when(j + 1 < nloc)
        def _():
            gdesc(bn).start()

        @pl.when(j + 2 < nloc)
        def _():
            r2 = row_of(j + 2)
            sdesc(b, r2).start()
            ddesc(b, r2).start()

        pltpu.sync_copy(rows[b], acc.at[dstb[b]], add=True)

    def pair(jj, carry):
        j = jj * 2
        step(j, 0)
        step(j + 1, 1)
        return carry

    lax.fori_loop(0, ROWS_PT // 2, pair, 0)

    @pl.when(nloc > ROWS_PT)
    def _():
        gdesc(0).wait()
        pltpu.sync_copy(rows[0], acc.at[dstb[0]], add=True)

    plsc.subcore_barrier()

    out_base = cid * N

    @pl.when(sid < NTILE - 1)
    def _():
        pltpu.sync_copy(acc.at[pl.ds(sid * WB, WB)],
                        agg_hbm.at[pl.ds(out_base + sid * WB, WB)])

    @pl.when(sid == NTILE - 1)
    def _():
        rest = N - (NTILE - 1) * WB
        pltpu.sync_copy(acc.at[pl.ds((NTILE - 1) * WB, rest)],
                        agg_hbm.at[pl.ds(out_base + (NTILE - 1) * WB, rest)])


def _segsum(table, src2, dst2, zeros):
    return pl.kernel(
        _segsum_body,
        mesh=plsc.VectorSubcoreMesh(core_axis_name="c", subcore_axis_name="s"),
        compiler_params=pltpu.CompilerParams(use_tc_tiling_on_sc=False),
        out_type=jax.ShapeDtypeStruct((2 * N, HW), jnp.float32),
        scratch_types=[
            pltpu.VMEM((CHUNK,), jnp.int32),
            pltpu.VMEM((CHUNK,), jnp.int32),
            pltpu.VMEM((CHUNK,), jnp.int32),
            pltpu.VMEM((CHUNK,), jnp.int32),
            pltpu.VMEM((CHUNK,), jnp.int32),
            pltpu.VMEM((CHUNK,), jnp.int32),
            pltpu.VMEM((CHUNK, HW), jnp.float32),
            pltpu.VMEM((CHUNK, HW), jnp.float32),
            pltpu.VMEM_SHARED((ACC_ROWS, HW), jnp.float32),
            pltpu.SemaphoreType.DMA,
            pltpu.SemaphoreType.DMA,
            pltpu.SemaphoreType.DMA,
            pltpu.SemaphoreType.DMA,
        ],
    )(table, src2, dst2, zeros)


def _seed_body(f_ref, w_ref, b_ref, h_ref):
    h_ref[...] = f_ref[...] * w_ref[...] + b_ref[...]


def _pass1_body(h_ref, agg_ref, g_ref, gb_ref, l_ref, lb_ref, z_ref, s_ref, q_ref):
    x = h_ref[...] + jnp.concatenate([agg_ref[0], agg_ref[1]], axis=-1)
    z = jnp.dot(x, g_ref[...], preferred_element_type=jnp.float32) + gb_ref[...]
    z = jnp.dot(z, l_ref[...], preferred_element_type=jnp.float32) + lb_ref[...]
    z_ref[...] = z

    @pl.when(pl.program_id(0) == 0)
    def _():
        s_ref[...] = jnp.zeros_like(s_ref)
        q_ref[...] = jnp.zeros_like(q_ref)

    s_ref[...] += jnp.sum(z, axis=0, keepdims=True)
    q_ref[...] += jnp.sum(z * z, axis=0, keepdims=True)


def _pass2_body(z_ref, s_ref, q_ref, g_ref, b_ref, h_ref, ro_ref):
    mean = s_ref[...] * (1.0 / N)
    var = q_ref[...] * (1.0 / N) - mean * mean
    inv = lax.rsqrt(var + 1e-5)
    scale = g_ref[...] * inv
    shift = b_ref[...] - mean * scale
    hn = jnp.maximum(z_ref[...] * scale + shift, 0.0)
    h_ref[...] = hn
    ro_ref[...] = hn.reshape(GROUPS, FLOW_LEN, R).sum(axis=1)[None]


def _cls_body(r0_ref, r1_ref, r2_ref, w0_ref, w1_ref, w2_ref, b_ref, y_ref):
    y = jnp.dot(r0_ref[...], w0_ref[...], preferred_element_type=jnp.float32)
    y += jnp.dot(r1_ref[...], w1_ref[...], preferred_element_type=jnp.float32)
    y += jnp.dot(r2_ref[...], w2_ref[...], preferred_element_type=jnp.float32)
    y_ref[...] = y + b_ref[...]


def kernel(feats, edge_index, W_seq, b_seq, gin_W, gin_b, lin_W, lin_b, bn_g, bn_b, cls_W, cls_b):
    f32 = jnp.float32
    src = edge_index[0].astype(jnp.int32)
    dst = edge_index[1].astype(jnp.int32)
    src2 = src.reshape(EROWS, CHUNK)
    dst2 = dst.reshape(EROWS, CHUNK)
    zeros32 = jnp.zeros((Z32, HW), f32)

    h = pl.pallas_call(
        _seed_body,
        grid=(GRID,),
        in_specs=[
            pl.BlockSpec((BLK, 1), lambda i: (i, 0)),
            pl.BlockSpec((1, R), lambda i: (0, 0)),
            pl.BlockSpec((1, R), lambda i: (0, 0)),
        ],
        out_specs=pl.BlockSpec((BLK, R), lambda i: (i, 0)),
        out_shape=jax.ShapeDtypeStruct((N, R), f32),
    )(feats.reshape(N, 1), W_seq, b_seq.reshape(1, R))

    ros = []
    for i in range(0, ORDER):
        agg = _segsum(h.reshape(2 * N, HW), src2, dst2, zeros32).reshape(2, N, HW)

        z, s, q = pl.pallas_call(
            _pass1_body,
            grid=(GRID,),
            in_specs=[
                pl.BlockSpec((BLK, R), lambda i: (i, 0)),
                pl.BlockSpec((2, BLK, HW), lambda i: (0, i, 0)),
                pl.BlockSpec((R, R), lambda i: (0, 0)),
                pl.BlockSpec((1, R), lambda i: (0, 0)),
                pl.BlockSpec((R, R), lambda i: (0, 0)),
                pl.BlockSpec((1, R), lambda i: (0, 0)),
            ],
            out_specs=[
                pl.BlockSpec((BLK, R), lambda i: (i, 0)),
                pl.BlockSpec((1, R), lambda i: (0, 0)),
                pl.BlockSpec((1, R), lambda i: (0, 0)),
            ],
            out_shape=[
                jax.ShapeDtypeStruct((N, R), f32),
                jax.ShapeDtypeStruct((1, R), f32),
                jax.ShapeDtypeStruct((1, R), f32),
            ],
        )(h, agg, gin_W[i], gin_b[i].reshape(1, R), lin_W[i], lin_b[i].reshape(1, R))

        h, ro = pl.pallas_call(
            _pass2_body,
            grid=(GRID,),
            in_specs=[
                pl.BlockSpec((BLK, R), lambda i: (i, 0)),
                pl.BlockSpec((1, R), lambda i: (0, 0)),
                pl.BlockSpec((1, R), lambda i: (0, 0)),
                pl.BlockSpec((1, R), lambda i: (0, 0)),
                pl.BlockSpec((1, R), lambda i: (0, 0)),
            ],
            out_specs=[
                pl.BlockSpec((BLK, R), lambda i: (i, 0)),
                pl.BlockSpec((1, GROUPS, R), lambda i: (i, 0, 0)),
            ],
            out_shape=[
                jax.ShapeDtypeStruct((N, R), f32),
                jax.ShapeDtypeStruct((GRID, GROUPS, R), f32),
            ],
        )(z, s, q, bn_g[i].reshape(1, R), bn_b[i].reshape(1, R))
        ros.append(ro.reshape(BS, R))

    wp = jnp.pad(cls_W, ((0, 0), (0, 128 - C)))
    bp = jnp.pad(cls_b, (0, 128 - C)).reshape(1, 128)
    y = pl.pallas_call(
        _cls_body,
        out_shape=jax.ShapeDtypeStruct((BS, 128), f32),
    )(ros[0], ros[1], ros[2], wp[0:R], wp[R:2 * R], wp[2 * R:3 * R], bp)
    return y[:, :C]
